# baseline (device time: 17614 ns/iter reference)
import jax
import jax.numpy as jnp
from jax import lax
from jax.experimental import pallas as pl
from jax.experimental.pallas import tpu as pltpu

X_SIZE = 2


def kernel(x):
    m_per, n = x.shape
    half = n // X_SIZE

    def body(x_ref, out_ref, send_sem, recv_sem):
        my_x = lax.axis_index("x")
        my_y = lax.axis_index("y")
        my_z = lax.axis_index("z")
        peer = (1 - my_x, my_y, my_z)

        barrier_sem = pltpu.get_barrier_semaphore()
        pl.semaphore_signal(
            barrier_sem, inc=1, device_id=peer,
            device_id_type=pl.DeviceIdType.MESH,
        )
        pl.semaphore_wait(barrier_sem, 1)

        def exchange(me):
            other = 1 - me
            rdma = pltpu.make_async_remote_copy(
                src_ref=x_ref.at[:, pl.ds(other * half, half)],
                dst_ref=out_ref.at[pl.ds(me * m_per, m_per), :],
                send_sem=send_sem,
                recv_sem=recv_sem,
                device_id=peer,
                device_id_type=pl.DeviceIdType.MESH,
            )
            rdma.start()
            out_ref[pl.ds(me * m_per, m_per), :] = x_ref[:, pl.ds(me * half, half)]
            rdma.wait()

        @pl.when(my_x == 0)
        def _():
            exchange(0)

        @pl.when(my_x == 1)
        def _():
            exchange(1)

    return pl.pallas_call(
        body,
        out_shape=jax.ShapeDtypeStruct((X_SIZE * m_per, half), x.dtype),
        in_specs=[pl.BlockSpec(memory_space=pltpu.VMEM)],
        out_specs=pl.BlockSpec(memory_space=pltpu.VMEM),
        scratch_shapes=[
            pltpu.SemaphoreType.DMA,
            pltpu.SemaphoreType.DMA,
        ],
        compiler_params=pltpu.CompilerParams(collective_id=0),
    )(x)


# device time: 17605 ns/iter; 1.0005x vs baseline; 1.0005x over previous
import jax
import jax.numpy as jnp
from jax import lax
from jax.experimental import pallas as pl
from jax.experimental.pallas import tpu as pltpu

X_SIZE = 2


def kernel(x):
    m_per, n = x.shape
    half = n // X_SIZE

    def body(x_ref, out_ref, local_sem, send_sem, recv_sem):
        my_x = lax.axis_index("x")
        my_y = lax.axis_index("y")
        my_z = lax.axis_index("z")
        peer = (1 - my_x, my_y, my_z)

        barrier_sem = pltpu.get_barrier_semaphore()
        pl.semaphore_signal(
            barrier_sem, inc=1, device_id=peer,
            device_id_type=pl.DeviceIdType.MESH,
        )
        pl.semaphore_wait(barrier_sem, 1)

        rdma = pltpu.make_async_remote_copy(
            src_ref=x_ref.at[:, pl.ds((1 - my_x) * half, half)],
            dst_ref=out_ref.at[pl.ds(my_x * m_per, m_per), :],
            send_sem=send_sem,
            recv_sem=recv_sem,
            device_id=peer,
            device_id_type=pl.DeviceIdType.MESH,
        )
        rdma.start()

        keep = pltpu.make_async_copy(
            x_ref.at[:, pl.ds(my_x * half, half)],
            out_ref.at[pl.ds(my_x * m_per, m_per), :],
            local_sem,
        )
        keep.start()
        keep.wait()

        rdma.wait()

    return pl.pallas_call(
        body,
        out_shape=jax.ShapeDtypeStruct((X_SIZE * m_per, half), x.dtype),
        in_specs=[pl.BlockSpec(memory_space=pltpu.HBM)],
        out_specs=pl.BlockSpec(memory_space=pltpu.HBM),
        scratch_shapes=[
            pltpu.SemaphoreType.DMA,
            pltpu.SemaphoreType.DMA,
            pltpu.SemaphoreType.DMA,
        ],
        compiler_params=pltpu.CompilerParams(collective_id=0),
    )(x)
